# baseline (speedup 1.0000x reference)
import functools

import jax
import jax.numpy as jnp
import numpy as np
from jax import lax
from jax.experimental import pallas as pl
from jax.experimental.pallas import tpu as pltpu
from jax.experimental.pallas import tpu_sc as plsc

B = 8
C = 21
CP = 32
CPO = 128
N = 1024
OVER = 3 * N
N_IMP = 768
SZ = 4096
C2 = 512

_CONSTS = {}


def _rotl(x, d):
    return ((x << np.uint32(d)) | (x >> np.uint32(32 - d))).astype(np.uint32)


def _threefry2x32(k0, k1, x0, x1):
    rot = (13, 15, 26, 6, 17, 29, 16, 24)
    ks0 = np.uint32(k0)
    ks1 = np.uint32(k1)
    ks2 = np.uint32(ks0 ^ ks1 ^ np.uint32(0x1BD11BDA))
    ks = (ks0, ks1, ks2)
    x0 = (x0 + ks0).astype(np.uint32)
    x1 = (x1 + ks1).astype(np.uint32)
    for i in range(5):
        for r in rot[0:4] if i % 2 == 0 else rot[4:8]:
            x0 = (x0 + x1).astype(np.uint32)
            x1 = (_rotl(x1, r) ^ x0).astype(np.uint32)
        x0 = (x0 + ks[(i + 1) % 3]).astype(np.uint32)
        x1 = (x1 + ks[(i + 2) % 3] + np.uint32(i + 1)).astype(np.uint32)
    return x0, x1


def _np_bits32(keypair, n):
    b1, b2 = _threefry2x32(keypair[0], keypair[1],
                           np.zeros(n, np.uint32), np.arange(n, dtype=np.uint32))
    return (b1 ^ b2).astype(np.uint32)


def _np_split(keypair):
    b1, b2 = _threefry2x32(keypair[0], keypair[1],
                           np.zeros(2, np.uint32), np.arange(2, dtype=np.uint32))
    return np.stack([b1, b2], axis=1)


def _np_randint(keypair, shape, lo, hi):
    n = int(np.prod(shape))
    sub = _np_split(keypair)
    hi_bits = _np_bits32(sub[0], n)
    lo_bits = _np_bits32(sub[1], n)
    span = np.uint32(hi - lo)
    mult = np.uint32((np.uint64(65536 % int(span)) ** 2) % np.uint64(span))
    val = ((hi_bits % span) * mult + (lo_bits % span)) % span
    return (np.int32(lo) + val.astype(np.int32)).reshape(shape)


def _consts():
    if not _CONSTS:
        sub = _np_split(np.array([0, 1234], dtype=np.uint32))
        idx = _np_randint(sub[0], (B, OVER), 0, N)
        cov = _np_randint(sub[1], (B, N - N_IMP), 0, N)
        pos = np.arange(SZ, dtype=np.int64)
        point_c = np.concatenate([idx, np.zeros((B, SZ - OVER), np.int32)], axis=1)
        payload = ((pos[None, :] << 10) | point_c).astype(np.int32)
        _CONSTS.update(idx=idx, cov=cov, payload=payload)
    return _CONSTS


def _dense_body(o_ref, r_ref, wc_ref, wf_ref, bias_ref, u_ref, d_ref):
    o = o_ref[...]
    r = r_ref[...]
    m1 = jnp.max(o, axis=0)
    is_max = o == m1[None, :]
    nmax = jnp.sum(is_max.astype(jnp.float32), axis=0)
    m2 = jnp.where(nmax > 1.0, m1,
                   jnp.max(jnp.where(is_max, -jnp.inf, o), axis=0))
    u_ref[...] = (-(m1 - m2))[None, :]

    dn = (((0,), (1,)), ((), ()))
    d = lax.dot_general(o, wc_ref[...], dn, preferred_element_type=jnp.float32)
    d += lax.dot_general(r, wf_ref[...], dn, preferred_element_type=jnp.float32)
    d_ref[...] = d + bias_ref[...]


def _dense_call(o_flat, r2s, wcp, wfp, biasp, interpret=False):
    return pl.pallas_call(
        _dense_body,
        grid=(B, N // 128),
        in_specs=[
            pl.BlockSpec((None, C, 128), lambda b, i: (b, 0, i)),
            pl.BlockSpec((None, None, C2, 128), lambda b, i: (b, i, 0, 0)),
            pl.BlockSpec((CPO, C), lambda b, i: (0, 0)),
            pl.BlockSpec((CPO, C2), lambda b, i: (0, 0)),
            pl.BlockSpec((1, CPO), lambda b, i: (0, 0)),
        ],
        out_specs=[
            pl.BlockSpec((None, 1, 128), lambda b, i: (b, 0, i)),
            pl.BlockSpec((None, 128, CPO), lambda b, i: (b, i, 0)),
        ],
        out_shape=[
            jax.ShapeDtypeStruct((B, 1, N), jnp.float32),
            jax.ShapeDtypeStruct((B, N, CPO), jnp.float32),
        ],
        interpret=interpret,
    )(o_flat, r2s, wcp, wfp, biasp)


def _sort_body(v_ref, p_ref, cov_ref, pts_ref):
    us = v_ref[...]
    v = jnp.concatenate(
        [us, jnp.full((B, SZ - OVER), -jnp.inf, jnp.float32)], axis=1)
    p = p_ref[...]
    iota = lax.broadcasted_iota(jnp.int32, (B, SZ), 1)
    for m in range(1, 13):
        k = 1 << m
        blk = (iota & k) == 0
        for j in reversed(range(m)):
            s = 1 << j
            lower = (iota & s) == 0
            pv = jnp.where(lower, pltpu.roll(v, SZ - s, 1), pltpu.roll(v, s, 1))
            pp = jnp.where(lower, pltpu.roll(p, SZ - s, 1), pltpu.roll(p, s, 1))
            less = (v > pv) | ((v == pv) & (p < pp))
            take_self = less == (blk == lower)
            v = jnp.where(take_self, v, pv)
            p = jnp.where(take_self, p, pp)
    pts_ref[...] = jnp.concatenate(
        [p[:, :N_IMP] & (N - 1), cov_ref[...]], axis=1)


def _sort_call(v, payload, cov, interpret=False):
    return pl.pallas_call(
        _sort_body,
        out_shape=jax.ShapeDtypeStruct((B, N), jnp.int32),
        interpret=interpret,
    )(v, payload, cov)


def _make_gather_u():
    mesh = plsc.VectorSubcoreMesh(core_axis_name="c", subcore_axis_name="s")
    chunk = (B * OVER) // 32

    @functools.partial(
        pl.kernel, mesh=mesh,
        out_type=jax.ShapeDtypeStruct((B, OVER), jnp.float32),
        compiler_params=pltpu.CompilerParams(needs_layout_passes=False),
        scratch_types=[
            pltpu.VMEM((N,), jnp.float32),
            pltpu.VMEM((chunk,), jnp.int32),
            pltpu.VMEM((chunk,), jnp.float32),
        ],
    )
    def gather_u(u_hbm, idx_hbm, out_hbm, u_v, idx_v, out_v):
        wid = lax.axis_index("s") * 2 + lax.axis_index("c")
        b = wid // 4
        q = wid % 4
        pltpu.sync_copy(u_hbm.at[b], u_v)
        pltpu.sync_copy(idx_hbm.at[b, pl.ds(q * chunk, chunk)], idx_v)
        for i in range(chunk // 16):
            sl = pl.ds(i * 16, 16)
            out_v[sl] = plsc.load_gather(u_v, [idx_v[sl]])
        pltpu.sync_copy(out_v, out_hbm.at[b, pl.ds(q * chunk, chunk)])

    return gather_u


def _make_gather_rows():
    mesh = plsc.VectorSubcoreMesh(core_axis_name="c", subcore_axis_name="s")
    npts = N // 4
    nchk = npts // 128

    @functools.partial(
        pl.kernel, mesh=mesh,
        out_type=jax.ShapeDtypeStruct((B, N, CPO), jnp.float32),
        compiler_params=pltpu.CompilerParams(needs_layout_passes=False),
        scratch_types=[
            pltpu.VMEM((nchk, 128), jnp.int32),
            pltpu.VMEM((128, CPO), jnp.float32),
            pltpu.SemaphoreType.DMA,
        ],
    )
    def gather_rows(dt_hbm, pts_hbm, out_hbm, pts_v, rows_v, sem):
        wid = lax.axis_index("s") * 2 + lax.axis_index("c")
        b = wid // 4
        q = wid % 4
        for j in range(nchk):
            pltpu.sync_copy(
                pts_hbm.at[b, pl.ds(q * npts + j * 128, 128)], pts_v.at[j])
        for j in range(nchk):
            pltpu.async_copy(dt_hbm.at[b].at[pts_v.at[j]], rows_v, sem).wait()
            pltpu.sync_copy(
                rows_v, out_hbm.at[b].at[pl.ds(q * npts + j * 128, 128)])

    return gather_rows


def kernel(x, res2, out, W, b):
    cst = _consts()
    idx_c = jnp.asarray(cst["idx"])
    cov_c = jnp.asarray(cst["cov"])
    payload_c = jnp.asarray(cst["payload"])

    out_flat = out.reshape(B, C, N)
    r2s = jnp.swapaxes(res2[:, :, :N // 128, :], 1, 2)
    w2 = W[:, :, 0]
    wcp = jnp.pad(w2[:, :C], ((0, CPO - C), (0, 0)))
    wfp = jnp.pad(w2[:, C:], ((0, CPO - C), (0, 0)))
    biasp = jnp.pad(b, (0, CPO - C)).reshape(1, CPO)

    u, d = _dense_call(out_flat, r2s, wcp, wfp, biasp)

    us = _make_gather_u()(u.reshape(B, N), idx_c)
    points = _sort_call(us, payload_c, cov_c)

    rend_t = _make_gather_rows()(d, points)
    rend = jnp.transpose(rend_t, (0, 2, 1))[:, :C, :]
    return rend, points


# speedup vs baseline: 1.3533x; 1.3533x over previous
import functools

import jax
import jax.numpy as jnp
import numpy as np
from jax import lax
from jax.experimental import pallas as pl
from jax.experimental.pallas import tpu as pltpu
from jax.experimental.pallas import tpu_sc as plsc

B = 8
C = 21
CP = 32
CPO = 128
N = 1024
OVER = 3 * N
N_IMP = 768
SZ = 4096
C2 = 512

_CONSTS = {}


def _rotl(x, d):
    return ((x << np.uint32(d)) | (x >> np.uint32(32 - d))).astype(np.uint32)


def _threefry2x32(k0, k1, x0, x1):
    rot = (13, 15, 26, 6, 17, 29, 16, 24)
    ks0 = np.uint32(k0)
    ks1 = np.uint32(k1)
    ks2 = np.uint32(ks0 ^ ks1 ^ np.uint32(0x1BD11BDA))
    ks = (ks0, ks1, ks2)
    x0 = (x0 + ks0).astype(np.uint32)
    x1 = (x1 + ks1).astype(np.uint32)
    for i in range(5):
        for r in rot[0:4] if i % 2 == 0 else rot[4:8]:
            x0 = (x0 + x1).astype(np.uint32)
            x1 = (_rotl(x1, r) ^ x0).astype(np.uint32)
        x0 = (x0 + ks[(i + 1) % 3]).astype(np.uint32)
        x1 = (x1 + ks[(i + 2) % 3] + np.uint32(i + 1)).astype(np.uint32)
    return x0, x1


def _np_bits32(keypair, n):
    b1, b2 = _threefry2x32(keypair[0], keypair[1],
                           np.zeros(n, np.uint32), np.arange(n, dtype=np.uint32))
    return (b1 ^ b2).astype(np.uint32)


def _np_split(keypair):
    b1, b2 = _threefry2x32(keypair[0], keypair[1],
                           np.zeros(2, np.uint32), np.arange(2, dtype=np.uint32))
    return np.stack([b1, b2], axis=1)


def _np_randint(keypair, shape, lo, hi):
    n = int(np.prod(shape))
    sub = _np_split(keypair)
    hi_bits = _np_bits32(sub[0], n)
    lo_bits = _np_bits32(sub[1], n)
    span = np.uint32(hi - lo)
    mult = np.uint32((np.uint64(65536 % int(span)) ** 2) % np.uint64(span))
    val = ((hi_bits % span) * mult + (lo_bits % span)) % span
    return (np.int32(lo) + val.astype(np.int32)).reshape(shape)


def _consts():
    if not _CONSTS:
        sub = _np_split(np.array([0, 1234], dtype=np.uint32))
        idx = _np_randint(sub[0], (B, OVER), 0, N)
        cov = _np_randint(sub[1], (B, N - N_IMP), 0, N)
        pos = np.arange(SZ, dtype=np.int64)
        point_c = np.concatenate([idx, np.zeros((B, SZ - OVER), np.int32)], axis=1)
        payload = ((pos[None, :] << 10) | point_c).astype(np.int32)
        _CONSTS.update(idx=idx, cov=cov, payload=payload)
    return _CONSTS


def _dense_body(o_ref, r_ref, wc_ref, wf_ref, bias_ref, u_ref, d_ref):
    o = o_ref[...]
    m1 = jnp.max(o, axis=0)
    is_max = o == m1[None, :]
    nmax = jnp.sum(is_max.astype(jnp.float32), axis=0)
    m2 = jnp.where(nmax > 1.0, m1,
                   jnp.max(jnp.where(is_max, -jnp.inf, o), axis=0))
    u_ref[...] = (-(m1 - m2))[None, :]

    dn = (((0,), (1,)), ((), ()))
    wc = wc_ref[...]
    wf = wf_ref[...]
    bias = bias_ref[...]
    for i in range(N // 128):
        d = lax.dot_general(o[:, i * 128:(i + 1) * 128], wc, dn,
                            preferred_element_type=jnp.float32)
        d += lax.dot_general(r_ref[i], wf, dn,
                             preferred_element_type=jnp.float32)
        d_ref[pl.ds(i * 128, 128), :] = d + bias


def _dense_call(o_flat, r2s, wcp, wfp, biasp, interpret=False):
    return pl.pallas_call(
        _dense_body,
        grid=(B,),
        in_specs=[
            pl.BlockSpec((None, C, N), lambda b: (b, 0, 0)),
            pl.BlockSpec((None, N // 128, C2, 128), lambda b: (b, 0, 0, 0)),
            pl.BlockSpec((CPO, C), lambda b: (0, 0)),
            pl.BlockSpec((CPO, C2), lambda b: (0, 0)),
            pl.BlockSpec((1, CPO), lambda b: (0, 0)),
        ],
        out_specs=[
            pl.BlockSpec((None, 1, N), lambda b: (b, 0, 0)),
            pl.BlockSpec((None, N, CPO), lambda b: (b, 0, 0)),
        ],
        out_shape=[
            jax.ShapeDtypeStruct((B, 1, N), jnp.float32),
            jax.ShapeDtypeStruct((B, N, CPO), jnp.float32),
        ],
        interpret=interpret,
    )(o_flat, r2s, wcp, wfp, biasp)


def _sort_body(v_ref, p_ref, cov_ref, pts_ref):
    us = v_ref[...]
    v = jnp.concatenate(
        [us, jnp.full((B, SZ - OVER), -jnp.inf, jnp.float32)], axis=1)
    p = p_ref[...]
    iota = lax.broadcasted_iota(jnp.int32, (B, SZ), 1)
    for m in range(1, 13):
        k = 1 << m
        blk = (iota & k) == 0
        for j in reversed(range(m)):
            s = 1 << j
            lower = (iota & s) == 0
            pv = jnp.where(lower, pltpu.roll(v, SZ - s, 1), pltpu.roll(v, s, 1))
            pp = jnp.where(lower, pltpu.roll(p, SZ - s, 1), pltpu.roll(p, s, 1))
            less = (v > pv) | ((v == pv) & (p < pp))
            take_self = less == (blk == lower)
            v = jnp.where(take_self, v, pv)
            p = jnp.where(take_self, p, pp)
    pts_ref[...] = jnp.concatenate(
        [p[:, :N_IMP] & (N - 1), cov_ref[...]], axis=1)


def _sort_call(v, payload, cov, interpret=False):
    return pl.pallas_call(
        _sort_body,
        out_shape=jax.ShapeDtypeStruct((B, N), jnp.int32),
        interpret=interpret,
    )(v, payload, cov)


def _make_gather_u():
    mesh = plsc.VectorSubcoreMesh(core_axis_name="c", subcore_axis_name="s")
    chunk = (B * OVER) // 32

    @functools.partial(
        pl.kernel, mesh=mesh,
        out_type=jax.ShapeDtypeStruct((B, OVER), jnp.float32),
        compiler_params=pltpu.CompilerParams(needs_layout_passes=False),
        scratch_types=[
            pltpu.VMEM((N,), jnp.float32),
            pltpu.VMEM((chunk,), jnp.int32),
            pltpu.VMEM((chunk,), jnp.float32),
        ],
    )
    def gather_u(u_hbm, idx_hbm, out_hbm, u_v, idx_v, out_v):
        wid = lax.axis_index("s") * 2 + lax.axis_index("c")
        b = wid // 4
        q = wid % 4
        pltpu.sync_copy(u_hbm.at[b], u_v)
        pltpu.sync_copy(idx_hbm.at[b, pl.ds(q * chunk, chunk)], idx_v)
        for i in range(chunk // 16):
            sl = pl.ds(i * 16, 16)
            out_v[sl] = plsc.load_gather(u_v, [idx_v[sl]])
        pltpu.sync_copy(out_v, out_hbm.at[b, pl.ds(q * chunk, chunk)])

    return gather_u


def _make_gather_rows():
    mesh = plsc.VectorSubcoreMesh(core_axis_name="c", subcore_axis_name="s")
    npts = N // 4
    nchk = npts // 128

    @functools.partial(
        pl.kernel, mesh=mesh,
        out_type=jax.ShapeDtypeStruct((B, N, CPO), jnp.float32),
        compiler_params=pltpu.CompilerParams(needs_layout_passes=False),
        scratch_types=[
            pltpu.VMEM((nchk, 128), jnp.int32),
            pltpu.VMEM((128, CPO), jnp.float32),
            pltpu.SemaphoreType.DMA,
        ],
    )
    def gather_rows(dt_hbm, pts_hbm, out_hbm, pts_v, rows_v, sem):
        wid = lax.axis_index("s") * 2 + lax.axis_index("c")
        b = wid // 4
        q = wid % 4
        for j in range(nchk):
            pltpu.sync_copy(
                pts_hbm.at[b, pl.ds(q * npts + j * 128, 128)], pts_v.at[j])
        for j in range(nchk):
            pltpu.async_copy(dt_hbm.at[b].at[pts_v.at[j]], rows_v, sem).wait()
            pltpu.sync_copy(
                rows_v, out_hbm.at[b].at[pl.ds(q * npts + j * 128, 128)])

    return gather_rows


def kernel(x, res2, out, W, b):
    cst = _consts()
    idx_c = jnp.asarray(cst["idx"])
    cov_c = jnp.asarray(cst["cov"])
    payload_c = jnp.asarray(cst["payload"])

    out_flat = out.reshape(B, C, N)
    r2s = jnp.swapaxes(res2[:, :, :N // 128, :], 1, 2)
    w2 = W[:, :, 0]
    wcp = jnp.pad(w2[:, :C], ((0, CPO - C), (0, 0)))
    wfp = jnp.pad(w2[:, C:], ((0, CPO - C), (0, 0)))
    biasp = jnp.pad(b, (0, CPO - C)).reshape(1, CPO)

    u, d = _dense_call(out_flat, r2s, wcp, wfp, biasp)

    us = _make_gather_u()(u.reshape(B, N), idx_c)
    points = _sort_call(us, payload_c, cov_c)

    rend_t = _make_gather_rows()(d, points)
    rend = jnp.transpose(rend_t, (0, 2, 1))[:, :C, :]
    return rend, points


# speedup vs baseline: 1.4428x; 1.0661x over previous
import functools

import jax
import jax.numpy as jnp
import numpy as np
from jax import lax
from jax.experimental import pallas as pl
from jax.experimental.pallas import tpu as pltpu
from jax.experimental.pallas import tpu_sc as plsc

B = 8
C = 21
CP = 32
CPO = 128
N = 1024
OVER = 3 * N
N_IMP = 768
SZ = 4096
C2 = 512

_CONSTS = {}


def _rotl(x, d):
    return ((x << np.uint32(d)) | (x >> np.uint32(32 - d))).astype(np.uint32)


def _threefry2x32(k0, k1, x0, x1):
    rot = (13, 15, 26, 6, 17, 29, 16, 24)
    ks0 = np.uint32(k0)
    ks1 = np.uint32(k1)
    ks2 = np.uint32(ks0 ^ ks1 ^ np.uint32(0x1BD11BDA))
    ks = (ks0, ks1, ks2)
    x0 = (x0 + ks0).astype(np.uint32)
    x1 = (x1 + ks1).astype(np.uint32)
    for i in range(5):
        for r in rot[0:4] if i % 2 == 0 else rot[4:8]:
            x0 = (x0 + x1).astype(np.uint32)
            x1 = (_rotl(x1, r) ^ x0).astype(np.uint32)
        x0 = (x0 + ks[(i + 1) % 3]).astype(np.uint32)
        x1 = (x1 + ks[(i + 2) % 3] + np.uint32(i + 1)).astype(np.uint32)
    return x0, x1


def _np_bits32(keypair, n):
    b1, b2 = _threefry2x32(keypair[0], keypair[1],
                           np.zeros(n, np.uint32), np.arange(n, dtype=np.uint32))
    return (b1 ^ b2).astype(np.uint32)


def _np_split(keypair):
    b1, b2 = _threefry2x32(keypair[0], keypair[1],
                           np.zeros(2, np.uint32), np.arange(2, dtype=np.uint32))
    return np.stack([b1, b2], axis=1)


def _np_randint(keypair, shape, lo, hi):
    n = int(np.prod(shape))
    sub = _np_split(keypair)
    hi_bits = _np_bits32(sub[0], n)
    lo_bits = _np_bits32(sub[1], n)
    span = np.uint32(hi - lo)
    mult = np.uint32((np.uint64(65536 % int(span)) ** 2) % np.uint64(span))
    val = ((hi_bits % span) * mult + (lo_bits % span)) % span
    return (np.int32(lo) + val.astype(np.int32)).reshape(shape)


def _consts():
    if not _CONSTS:
        sub = _np_split(np.array([0, 1234], dtype=np.uint32))
        idx = _np_randint(sub[0], (B, OVER), 0, N)
        cov = _np_randint(sub[1], (B, N - N_IMP), 0, N)
        pos = np.arange(SZ, dtype=np.int64)
        point_c = np.concatenate([idx, np.zeros((B, SZ - OVER), np.int32)], axis=1)
        payload = ((pos[None, :] << 10) | point_c).astype(np.int32)
        _CONSTS.update(idx=idx, cov=cov, payload=payload)
    return _CONSTS


def _unc_body(o_ref, u_ref):
    o3 = o_ref[...]
    m1 = jnp.max(o3, axis=1)
    is_max = o3 == m1[:, None, :]
    nmax = jnp.sum(is_max.astype(jnp.float32), axis=1)
    m2 = jnp.where(nmax > 1.0, m1,
                   jnp.max(jnp.where(is_max, -jnp.inf, o3), axis=1))
    u_ref[...] = -(m1 - m2)


def _unc_call(o2d, interpret=False):
    return pl.pallas_call(
        _unc_body,
        out_shape=jax.ShapeDtypeStruct((B, N), jnp.float32),
        interpret=interpret,
    )(o2d)


def _head_body(o_ref, r_ref, wc_ref, wf_ref, bias_ref, d_ref):
    o = o_ref[...]
    dn = (((0,), (1,)), ((), ()))
    wc = wc_ref[...]
    wf = wf_ref[...]
    bias = bias_ref[...]
    for i in range(N // 128):
        d = lax.dot_general(o[:, i * 128:(i + 1) * 128], wc, dn,
                            preferred_element_type=jnp.float32)
        d += lax.dot_general(r_ref[i], wf, dn,
                             preferred_element_type=jnp.float32)
        d_ref[pl.ds(i * 128, 128), :] = d + bias


def _head_call(o_flat, r2s, wcp, wfp, biasp, interpret=False):
    return pl.pallas_call(
        _head_body,
        grid=(B,),
        in_specs=[
            pl.BlockSpec((None, C, N), lambda b: (b, 0, 0)),
            pl.BlockSpec((None, N // 128, C2, 128), lambda b: (b, 0, 0, 0)),
            pl.BlockSpec((CPO, C), lambda b: (0, 0)),
            pl.BlockSpec((CPO, C2), lambda b: (0, 0)),
            pl.BlockSpec((1, CPO), lambda b: (0, 0)),
        ],
        out_specs=pl.BlockSpec((None, N, CPO), lambda b: (b, 0, 0)),
        out_shape=jax.ShapeDtypeStruct((B, N, CPO), jnp.float32),
        interpret=interpret,
    )(o_flat, r2s, wcp, wfp, biasp)


def _sort_body(v_ref, p_ref, cov_ref, pts_ref):
    us = v_ref[...]
    v = jnp.concatenate(
        [us, jnp.full((B, SZ - OVER), -jnp.inf, jnp.float32)], axis=1)
    p = p_ref[...]
    iota = lax.broadcasted_iota(jnp.int32, (B, SZ), 1)
    for m in range(1, 13):
        k = 1 << m
        blk = (iota & k) == 0
        for j in reversed(range(m)):
            s = 1 << j
            lower = (iota & s) == 0
            pv = jnp.where(lower, pltpu.roll(v, SZ - s, 1), pltpu.roll(v, s, 1))
            pp = jnp.where(lower, pltpu.roll(p, SZ - s, 1), pltpu.roll(p, s, 1))
            less = (v > pv) | ((v == pv) & (p < pp))
            take_self = less == (blk == lower)
            v = jnp.where(take_self, v, pv)
            p = jnp.where(take_self, p, pp)
    pts_ref[...] = jnp.concatenate(
        [p[:, :N_IMP] & (N - 1), cov_ref[...]], axis=1)


def _sort_call(v, payload, cov, interpret=False):
    return pl.pallas_call(
        _sort_body,
        out_shape=jax.ShapeDtypeStruct((B, N), jnp.int32),
        interpret=interpret,
    )(v, payload, cov)


def _make_gather_u():
    mesh = plsc.VectorSubcoreMesh(core_axis_name="c", subcore_axis_name="s")
    chunk = (B * OVER) // 32

    @functools.partial(
        pl.kernel, mesh=mesh,
        out_type=jax.ShapeDtypeStruct((B, OVER), jnp.float32),
        compiler_params=pltpu.CompilerParams(needs_layout_passes=False),
        scratch_types=[
            pltpu.VMEM((N,), jnp.float32),
            pltpu.VMEM((chunk,), jnp.int32),
            pltpu.VMEM((chunk,), jnp.float32),
        ],
    )
    def gather_u(u_hbm, idx_hbm, out_hbm, u_v, idx_v, out_v):
        wid = lax.axis_index("s") * 2 + lax.axis_index("c")
        b = wid // 4
        q = wid % 4
        pltpu.sync_copy(u_hbm.at[b], u_v)
        pltpu.sync_copy(idx_hbm.at[b, pl.ds(q * chunk, chunk)], idx_v)
        for i in range(chunk // 16):
            sl = pl.ds(i * 16, 16)
            out_v[sl] = plsc.load_gather(u_v, [idx_v[sl]])
        pltpu.sync_copy(out_v, out_hbm.at[b, pl.ds(q * chunk, chunk)])

    return gather_u


def _make_gather_rows():
    mesh = plsc.VectorSubcoreMesh(core_axis_name="c", subcore_axis_name="s")
    npts = N // 4
    nchk = npts // 128

    @functools.partial(
        pl.kernel, mesh=mesh,
        out_type=jax.ShapeDtypeStruct((B, N, CPO), jnp.float32),
        compiler_params=pltpu.CompilerParams(needs_layout_passes=False),
        scratch_types=[
            pltpu.VMEM((nchk, 128), jnp.int32),
            pltpu.VMEM((128, CPO), jnp.float32),
            pltpu.SemaphoreType.DMA,
        ],
    )
    def gather_rows(dt_hbm, pts_hbm, out_hbm, pts_v, rows_v, sem):
        wid = lax.axis_index("s") * 2 + lax.axis_index("c")
        b = wid // 4
        q = wid % 4
        for j in range(nchk):
            pltpu.sync_copy(
                pts_hbm.at[b, pl.ds(q * npts + j * 128, 128)], pts_v.at[j])
        for j in range(nchk):
            pltpu.async_copy(dt_hbm.at[b].at[pts_v.at[j]], rows_v, sem).wait()
            pltpu.sync_copy(
                rows_v, out_hbm.at[b].at[pl.ds(q * npts + j * 128, 128)])

    return gather_rows


def kernel(x, res2, out, W, b):
    cst = _consts()
    idx_c = jnp.asarray(cst["idx"])
    cov_c = jnp.asarray(cst["cov"])
    payload_c = jnp.asarray(cst["payload"])

    out_flat = out.reshape(B, C, N)
    r2s = jnp.swapaxes(res2[:, :, :N // 128, :], 1, 2)
    w2 = W[:, :, 0]
    wcp = jnp.pad(w2[:, :C], ((0, CPO - C), (0, 0)))
    wfp = jnp.pad(w2[:, C:], ((0, CPO - C), (0, 0)))
    biasp = jnp.pad(b, (0, CPO - C)).reshape(1, CPO)

    u = _unc_call(out_flat)
    d = _head_call(out_flat, r2s, wcp, wfp, biasp)

    us = _make_gather_u()(u, idx_c)
    points = _sort_call(us, payload_c, cov_c)

    rend_t = _make_gather_rows()(d, points)
    rend = jnp.transpose(rend_t, (0, 2, 1))[:, :C, :]
    return rend, points


# speedup vs baseline: 1.5110x; 1.0472x over previous
import functools

import jax
import jax.numpy as jnp
import numpy as np
from jax import lax
from jax.experimental import pallas as pl
from jax.experimental.pallas import tpu as pltpu
from jax.experimental.pallas import tpu_sc as plsc

B = 8
C = 21
CP = 32
CPO = 128
N = 1024
OVER = 3 * N
N_IMP = 768
SZ = 4096
C2 = 512

_CONSTS = {}


def _rotl(x, d):
    return ((x << np.uint32(d)) | (x >> np.uint32(32 - d))).astype(np.uint32)


def _threefry2x32(k0, k1, x0, x1):
    rot = (13, 15, 26, 6, 17, 29, 16, 24)
    ks0 = np.uint32(k0)
    ks1 = np.uint32(k1)
    ks2 = np.uint32(ks0 ^ ks1 ^ np.uint32(0x1BD11BDA))
    ks = (ks0, ks1, ks2)
    x0 = (x0 + ks0).astype(np.uint32)
    x1 = (x1 + ks1).astype(np.uint32)
    for i in range(5):
        for r in rot[0:4] if i % 2 == 0 else rot[4:8]:
            x0 = (x0 + x1).astype(np.uint32)
            x1 = (_rotl(x1, r) ^ x0).astype(np.uint32)
        x0 = (x0 + ks[(i + 1) % 3]).astype(np.uint32)
        x1 = (x1 + ks[(i + 2) % 3] + np.uint32(i + 1)).astype(np.uint32)
    return x0, x1


def _np_bits32(keypair, n):
    b1, b2 = _threefry2x32(keypair[0], keypair[1],
                           np.zeros(n, np.uint32), np.arange(n, dtype=np.uint32))
    return (b1 ^ b2).astype(np.uint32)


def _np_split(keypair):
    b1, b2 = _threefry2x32(keypair[0], keypair[1],
                           np.zeros(2, np.uint32), np.arange(2, dtype=np.uint32))
    return np.stack([b1, b2], axis=1)


def _np_randint(keypair, shape, lo, hi):
    n = int(np.prod(shape))
    sub = _np_split(keypair)
    hi_bits = _np_bits32(sub[0], n)
    lo_bits = _np_bits32(sub[1], n)
    span = np.uint32(hi - lo)
    mult = np.uint32((np.uint64(65536 % int(span)) ** 2) % np.uint64(span))
    val = ((hi_bits % span) * mult + (lo_bits % span)) % span
    return (np.int32(lo) + val.astype(np.int32)).reshape(shape)


def _consts():
    if not _CONSTS:
        sub = _np_split(np.array([0, 1234], dtype=np.uint32))
        idx = _np_randint(sub[0], (B, OVER), 0, N)
        cov = _np_randint(sub[1], (B, N - N_IMP), 0, N)
        pos = np.arange(SZ, dtype=np.int64)
        point_c = np.concatenate([idx, np.zeros((B, SZ - OVER), np.int32)], axis=1)
        payload = ((pos[None, :] << 10) | point_c).astype(np.int32)
        _CONSTS.update(idx=idx, cov=cov, payload=payload)
    return _CONSTS


def _unc_body(o_ref, u_ref):
    o3 = o_ref[...]
    m1 = jnp.max(o3, axis=1)
    is_max = o3 == m1[:, None, :]
    nmax = jnp.sum(is_max.astype(jnp.float32), axis=1)
    m2 = jnp.where(nmax > 1.0, m1,
                   jnp.max(jnp.where(is_max, -jnp.inf, o3), axis=1))
    u_ref[...] = -(m1 - m2)


def _unc_call(o2d, interpret=False):
    return pl.pallas_call(
        _unc_body,
        out_shape=jax.ShapeDtypeStruct((B, N), jnp.float32),
        interpret=interpret,
    )(o2d)


def _head_body(o_ref, r_ref, wc_ref, wf_ref, bias_ref, dep_ref, d_ref):
    del dep_ref
    o = o_ref[...]
    dn = (((0,), (1,)), ((), ()))
    wc = wc_ref[...]
    wf = wf_ref[...]
    bias = bias_ref[...]
    for i in range(N // 128):
        d = lax.dot_general(o[:, i * 128:(i + 1) * 128], wc, dn,
                            preferred_element_type=jnp.float32)
        d += lax.dot_general(r_ref[i], wf, dn,
                             preferred_element_type=jnp.float32)
        d_ref[pl.ds(i * 128, 128), :] = d + bias


def _head_call(o_flat, r2s, wcp, wfp, biasp, dep, interpret=False):
    return pl.pallas_call(
        _head_body,
        grid=(B,),
        in_specs=[
            pl.BlockSpec((None, C, N), lambda b: (b, 0, 0)),
            pl.BlockSpec((None, N // 128, C2, 128), lambda b: (b, 0, 0, 0)),
            pl.BlockSpec((CPO, C), lambda b: (0, 0)),
            pl.BlockSpec((CPO, C2), lambda b: (0, 0)),
            pl.BlockSpec((1, CPO), lambda b: (0, 0)),
            pl.BlockSpec((None, 1, N), lambda b: (b, 0, 0)),
        ],
        out_specs=pl.BlockSpec((None, N, CPO), lambda b: (b, 0, 0)),
        out_shape=jax.ShapeDtypeStruct((B, N, CPO), jnp.float32),
        interpret=interpret,
    )(o_flat, r2s, wcp, wfp, biasp, dep)


def _sort_body(v_ref, p_ref, cov_ref, pts_ref):
    us = v_ref[...]
    v = jnp.concatenate(
        [us, jnp.full((B, SZ - OVER), -jnp.inf, jnp.float32)], axis=1)
    p = p_ref[...]
    iota = lax.broadcasted_iota(jnp.int32, (B, SZ), 1)
    for m in range(1, 13):
        k = 1 << m
        blk = (iota & k) == 0
        for j in reversed(range(m)):
            s = 1 << j
            lower = (iota & s) == 0
            pv = jnp.where(lower, pltpu.roll(v, SZ - s, 1), pltpu.roll(v, s, 1))
            pp = jnp.where(lower, pltpu.roll(p, SZ - s, 1), pltpu.roll(p, s, 1))
            less = (v > pv) | ((v == pv) & (p < pp))
            take_self = less == (blk == lower)
            v = jnp.where(take_self, v, pv)
            p = jnp.where(take_self, p, pp)
    pts_ref[...] = jnp.concatenate(
        [p[:, :N_IMP] & (N - 1), cov_ref[...]], axis=1)


def _sort_call(v, payload, cov, interpret=False):
    return pl.pallas_call(
        _sort_body,
        out_shape=jax.ShapeDtypeStruct((B, N), jnp.int32),
        interpret=interpret,
    )(v, payload, cov)


def _make_gather_u():
    mesh = plsc.VectorSubcoreMesh(core_axis_name="c", subcore_axis_name="s")
    chunk = (B * OVER) // 32

    @functools.partial(
        pl.kernel, mesh=mesh,
        out_type=jax.ShapeDtypeStruct((B, OVER), jnp.float32),
        compiler_params=pltpu.CompilerParams(needs_layout_passes=False),
        scratch_types=[
            pltpu.VMEM((N,), jnp.float32),
            pltpu.VMEM((chunk,), jnp.int32),
            pltpu.VMEM((chunk,), jnp.float32),
        ],
    )
    def gather_u(u_hbm, idx_hbm, out_hbm, u_v, idx_v, out_v):
        wid = lax.axis_index("s") * 2 + lax.axis_index("c")
        b = wid // 4
        q = wid % 4
        pltpu.sync_copy(u_hbm.at[b], u_v)
        pltpu.sync_copy(idx_hbm.at[b, pl.ds(q * chunk, chunk)], idx_v)
        for i in range(chunk // 16):
            sl = pl.ds(i * 16, 16)
            out_v[sl] = plsc.load_gather(u_v, [idx_v[sl]])
        pltpu.sync_copy(out_v, out_hbm.at[b, pl.ds(q * chunk, chunk)])

    return gather_u


def _make_gather_rows():
    mesh = plsc.VectorSubcoreMesh(core_axis_name="c", subcore_axis_name="s")
    npts = N // 4
    nchk = npts // 128

    @functools.partial(
        pl.kernel, mesh=mesh,
        out_type=jax.ShapeDtypeStruct((B, N, CPO), jnp.float32),
        compiler_params=pltpu.CompilerParams(needs_layout_passes=False),
        scratch_types=[
            pltpu.VMEM((nchk, 128), jnp.int32),
            pltpu.VMEM((128, CPO), jnp.float32),
            pltpu.SemaphoreType.DMA,
        ],
    )
    def gather_rows(dt_hbm, pts_hbm, out_hbm, pts_v, rows_v, sem):
        wid = lax.axis_index("s") * 2 + lax.axis_index("c")
        b = wid // 4
        q = wid % 4
        for j in range(nchk):
            pltpu.sync_copy(
                pts_hbm.at[b, pl.ds(q * npts + j * 128, 128)], pts_v.at[j])
        for j in range(nchk):
            pltpu.async_copy(dt_hbm.at[b].at[pts_v.at[j]], rows_v, sem).wait()
            pltpu.sync_copy(
                rows_v, out_hbm.at[b].at[pl.ds(q * npts + j * 128, 128)])

    return gather_rows


def kernel(x, res2, out, W, b):
    cst = _consts()
    idx_c = jnp.asarray(cst["idx"])
    cov_c = jnp.asarray(cst["cov"])
    payload_c = jnp.asarray(cst["payload"])

    out_flat = out.reshape(B, C, N)
    r2s = jnp.swapaxes(res2[:, :, :N // 128, :], 1, 2)
    w2 = W[:, :, 0]
    wcp = jnp.pad(w2[:, :C], ((0, CPO - C), (0, 0)))
    wfp = jnp.pad(w2[:, C:], ((0, CPO - C), (0, 0)))
    biasp = jnp.pad(b, (0, CPO - C)).reshape(1, CPO)

    u = _unc_call(out_flat)
    us = _make_gather_u()(u, idx_c)
    points = _sort_call(us, payload_c, cov_c)

    d = _head_call(out_flat, r2s, wcp, wfp, biasp, points.reshape(B, 1, N))

    rend_t = _make_gather_rows()(d, points)
    rend = jnp.transpose(rend_t, (0, 2, 1))[:, :C, :]
    return rend, points
